# baseline (speedup 1.0000x reference)
import functools

import jax
import jax.numpy as jnp
from jax import lax
from jax.experimental import pallas as pl
from jax.experimental.pallas import tpu as pltpu
from jax.experimental.pallas import tpu_sc as plsc

D_VOCAB = 100000
D_MODEL = 2048
B_TOTAL = 4 * 4096

_info = plsc.get_sparse_core_info()
NC = _info.num_cores
NS = _info.num_subcores
NW = NC * NS
BPW = B_TOTAL // NW
CHUNK = 4
NCHUNK = BPW // CHUNK
NT = 64
NBUF_T = 4
GAHEAD_T = 2
NBUF_S = 2

_mesh = plsc.VectorSubcoreMesh(core_axis_name="c", subcore_axis_name="s")


@functools.partial(
    pl.kernel,
    out_type=jax.ShapeDtypeStruct((B_TOTAL, D_MODEL), jnp.float32),
    mesh=_mesh,
    scratch_types=[
        pltpu.VMEM((NCHUNK, CHUNK), jnp.int32),
        [pltpu.VMEM((CHUNK, D_MODEL), jnp.float32)] * NBUF_T,
        [pltpu.VMEM((CHUNK, D_MODEL), jnp.float32)] * NBUF_S,
        pltpu.VMEM_SHARED((NS, NBUF_S, CHUNK, D_MODEL), jnp.float32),
        [pltpu.SemaphoreType.DMA] * NBUF_T,
        [pltpu.SemaphoreType.DMA] * NBUF_T,
        [pltpu.SemaphoreType.DMA] * NBUF_S,
        [pltpu.SemaphoreType.DMA] * NBUF_S,
        [pltpu.SemaphoreType.DMA] * NBUF_S,
    ],
)
def _embed_sc(tok_hbm, table_hbm, out_hbm, idx_v, bufs, tbufs, shared,
              gsems, ssems, gsems2, csems2, ssems2):
    sid = lax.axis_index("s")
    wid = sid * NC + lax.axis_index("c")
    base = wid * BPW

    pltpu.sync_copy(tok_hbm.at[wid], idx_v)

    def sbuf(b):
        return shared.at[sid, b]

    def gather(g, b):
        pltpu.async_copy(table_hbm.at[idx_v.at[g]], bufs[b], gsems[b])

    def gwait(b):
        pltpu.make_async_copy(table_hbm.at[idx_v.at[0]], bufs[b], gsems[b]).wait()

    def astore(g, b):
        pltpu.async_copy(bufs[b], out_hbm.at[pl.ds(base + g * CHUNK, CHUNK)], ssems[b])

    def swait(b):
        pltpu.make_async_copy(bufs[b], out_hbm.at[pl.ds(base, CHUNK)], ssems[b]).wait()

    def gather2(g, b):
        pltpu.async_copy(table_hbm.at[idx_v.at[g]], tbufs[b], gsems2[b])

    def gwait2(b):
        pltpu.make_async_copy(table_hbm.at[idx_v.at[0]], tbufs[b], gsems2[b]).wait()

    def cstart2(b):
        pltpu.async_copy(tbufs[b], sbuf(b), csems2[b])

    def cwait2(b):
        pltpu.make_async_copy(tbufs[b], sbuf(b), csems2[b]).wait()

    def astore2(g, b):
        pltpu.async_copy(sbuf(b), out_hbm.at[pl.ds(base + g * CHUNK, CHUNK)], ssems2[b])

    def swait2(b):
        pltpu.make_async_copy(sbuf(b), out_hbm.at[pl.ds(base, CHUNK)], ssems2[b]).wait()

    for b in range(GAHEAD_T):
        gather(b, b)
    gather2(NT, 0)

    def body(i, carry):
        g0 = i * NBUF_T
        for b in range(NBUF_T):
            g = g0 + b
            bs = b % NBUF_S
            bsn = (bs + 1) % NBUF_S
            gwait(b)
            astore(g, b)

            gwait2(bs)

            @pl.when(g >= NBUF_S)
            def _():
                swait2(bs)
            cstart2(bs)

            @pl.when(g >= 1)
            def _():
                cwait2(bsn)
                astore2(NT + g - 1, bsn)

            bn = (b + GAHEAD_T) % NBUF_T

            @pl.when(g + GAHEAD_T < NT)
            def _():
                @pl.when(g + GAHEAD_T >= NBUF_T)
                def _():
                    swait(bn)
                gather(g + GAHEAD_T, bn)

            @pl.when(NT + g + 1 < NCHUNK)
            def _():
                gather2(NT + g + 1, bsn)
        return carry

    lax.fori_loop(0, NT // NBUF_T, body, 0)
    for b in range(NBUF_T):
        swait(b)
    last_bs = (NT - 1) % NBUF_S
    cwait2(last_bs)
    astore2(NCHUNK - 1, last_bs)
    swait2((last_bs + 1) % NBUF_S)
    swait2(last_bs)


R_TC = 8


def _tc_body(tok_ref, *refs):
    del tok_ref
    ins = refs[:R_TC]
    out = refs[R_TC]
    for k in range(R_TC):
        out[k, :] = ins[k][0, 0, :]


def _embed_tc(tok, W_E):
    n = tok.shape[0]
    w3 = W_E.reshape(D_VOCAB, 1, D_MODEL)
    grid_spec = pltpu.PrefetchScalarGridSpec(
        num_scalar_prefetch=1,
        grid=(n // R_TC,),
        in_specs=[
            pl.BlockSpec(
                (1, 1, D_MODEL),
                functools.partial(
                    lambda k, i, tok_ref: (tok_ref[i * R_TC + k], 0, 0), k
                ),
            )
            for k in range(R_TC)
        ],
        out_specs=pl.BlockSpec((R_TC, D_MODEL), lambda i, tok_ref: (i, 0)),
    )
    return pl.pallas_call(
        _tc_body,
        grid_spec=grid_spec,
        out_shape=jax.ShapeDtypeStruct((n, D_MODEL), jnp.float32),
        compiler_params=pltpu.CompilerParams(
            dimension_semantics=("arbitrary",),
        ),
    )(tok, *([w3] * R_TC))


def kernel(tokens, W_E):
    tok = tokens.reshape(-1).astype(jnp.int32).reshape(NW, NCHUNK, CHUNK)
    out = _embed_sc(tok, W_E)
    return out.reshape(tokens.shape + (W_E.shape[1],)), tokens


# speedup vs baseline: 1.0875x; 1.0875x over previous
import functools

import jax
import jax.numpy as jnp
from jax import lax
from jax.experimental import pallas as pl
from jax.experimental.pallas import tpu as pltpu
from jax.experimental.pallas import tpu_sc as plsc

D_VOCAB = 100000
D_MODEL = 2048
B_TOTAL = 4 * 4096

_info = plsc.get_sparse_core_info()
NC = _info.num_cores
NS = _info.num_subcores
NW = NC * NS
BPW = B_TOTAL // NW
CHUNK = 8
NBUF = 4
NCHUNK = BPW // CHUNK
GAHEAD = 2

_mesh = plsc.VectorSubcoreMesh(core_axis_name="c", subcore_axis_name="s")


@functools.partial(
    pl.kernel,
    out_type=jax.ShapeDtypeStruct((B_TOTAL, D_MODEL), jnp.float32),
    mesh=_mesh,
    scratch_types=[
        pltpu.VMEM((NCHUNK, CHUNK), jnp.int32),
        [pltpu.VMEM((CHUNK, D_MODEL), jnp.float32)] * NBUF,
        [pltpu.SemaphoreType.DMA] * NBUF,
        [pltpu.SemaphoreType.DMA] * NBUF,
    ],
)
def _embed_sc(tok_hbm, table_hbm, out_hbm, idx_v, bufs, gsems, ssems):
    wid = lax.axis_index("s") * NC + lax.axis_index("c")
    base = wid * BPW

    pltpu.sync_copy(tok_hbm.at[wid], idx_v)

    def gather(g, b):
        pltpu.async_copy(table_hbm.at[idx_v.at[g]], bufs[b], gsems[b])

    def gwait(b):
        pltpu.make_async_copy(table_hbm.at[idx_v.at[0]], bufs[b], gsems[b]).wait()

    def astore(g, b):
        pltpu.async_copy(bufs[b], out_hbm.at[pl.ds(base + g * CHUNK, CHUNK)], ssems[b])

    def swait(b):
        pltpu.make_async_copy(bufs[b], out_hbm.at[pl.ds(base, CHUNK)], ssems[b]).wait()

    for b in range(GAHEAD):
        gather(b, b)

    def body(i, carry):
        g0 = i * NBUF
        for b in range(NBUF):
            g = g0 + b
            gwait(b)
            astore(g, b)
            bn = (b + GAHEAD) % NBUF

            @pl.when(g + GAHEAD < NCHUNK)
            def _():
                @pl.when(g + GAHEAD >= NBUF)
                def _():
                    swait(bn)
                gather(g + GAHEAD, bn)
        return carry

    lax.fori_loop(0, NCHUNK // NBUF, body, 0)
    for b in range(NBUF):
        swait(b)


def kernel(tokens, W_E):
    tok = tokens.reshape(-1).astype(jnp.int32).reshape(NW, NCHUNK, CHUNK)
    out = _embed_sc(tok, W_E)
    return out.reshape(tokens.shape + (W_E.shape[1],)), tokens
